# baseline (speedup 1.0000x reference)
import functools

import jax
import jax.numpy as jnp
from jax import lax
from jax.experimental import pallas as pl
from jax.experimental.pallas import tpu as pltpu
from jax.experimental.pallas import tpu_sc as plsc

B = 16384
ED = 64
FC = 102
FL = 23

_NEG_SLOPE = 0.01


def _make_gather(rows: int, d: int, b: int):
    info = plsc.get_sparse_core_info()
    nw = info.num_cores * info.num_subcores
    assert d % info.num_lanes == 0 and b % (8 * nw) == 0
    b_per_w = b // nw
    mesh = plsc.VectorSubcoreMesh(core_axis_name="c", subcore_axis_name="s")

    @functools.partial(
        pl.kernel,
        mesh=mesh,
        out_type=jax.ShapeDtypeStruct((b, d), jnp.float32),
        scratch_types=[
            pltpu.VMEM((b_per_w,), jnp.int32),
            pltpu.VMEM((b_per_w, d), jnp.float32),
            pltpu.SemaphoreType.DMA,
        ],
    )
    def gather_kernel(table_hbm, idx_hbm, out_hbm, idx_v, rows_v, sem):
        wid = lax.axis_index("s") * info.num_cores + lax.axis_index("c")
        base = wid * b_per_w
        pltpu.sync_copy(idx_hbm.at[pl.ds(base, b_per_w)], idx_v)
        pltpu.async_copy(table_hbm.at[idx_v], rows_v, sem).wait()
        pltpu.sync_copy(rows_v, out_hbm.at[pl.ds(base, b_per_w)])

    return gather_kernel


def _leaky(x):
    return jnp.where(x >= 0, x, _NEG_SLOPE * x)


def _mlp_body(fc_ref, fl_ref, pair_ref, par_ref, wd_ref, bd_ref, wc_ref,
              wl_ref, we_ref, bs_ref, wcls_ref, bcls_ref,
              out_ref, wide_ref, ef_ref):
    fc = fc_ref[...]
    fl = fl_ref[...]
    pair = pair_ref[...]
    odd = par_ref[...] > 0
    er = jnp.where(odd, pair[:, ED:], pair[:, :ED])
    ef = _leaky(jnp.dot(er, wd_ref[...], preferred_element_type=jnp.float32)
                + bd_ref[...])
    h = (jnp.dot(fc, wc_ref[...], preferred_element_type=jnp.float32)
         + jnp.dot(fl, wl_ref[...], preferred_element_type=jnp.float32)
         + jnp.dot(ef, we_ref[...], preferred_element_type=jnp.float32)
         + bs_ref[...])
    h = _leaky(h)
    out_ref[...] = jnp.dot(h, wcls_ref[...], preferred_element_type=jnp.float32) + bcls_ref[...]
    wide_ref[...] = jnp.concatenate([fc, fl], axis=1)
    ef_ref[...] = ef


def _mlp(fc, fl, pair, par, wd_t, bd, wc_t, wl_t, we_t, bs, wcls_t, bcls,
         tile: int = 2048):
    grid = (B // tile,)
    row = lambda i: (i, 0)
    rep = lambda i: (0, 0)
    return pl.pallas_call(
        _mlp_body,
        grid=grid,
        in_specs=[
            pl.BlockSpec((tile, FC), row),
            pl.BlockSpec((tile, FL), row),
            pl.BlockSpec((tile, 2 * ED), row),
            pl.BlockSpec((tile, 1), row),
            pl.BlockSpec(wd_t.shape, rep),
            pl.BlockSpec(bd.shape, rep),
            pl.BlockSpec(wc_t.shape, rep),
            pl.BlockSpec(wl_t.shape, rep),
            pl.BlockSpec(we_t.shape, rep),
            pl.BlockSpec(bs.shape, rep),
            pl.BlockSpec(wcls_t.shape, rep),
            pl.BlockSpec(bcls.shape, rep),
        ],
        out_specs=[
            pl.BlockSpec((tile, 2), row),
            pl.BlockSpec((tile, FC + FL), row),
            pl.BlockSpec((tile, ED), row),
        ],
        out_shape=[
            jax.ShapeDtypeStruct((B, 2), jnp.float32),
            jax.ShapeDtypeStruct((B, FC + FL), jnp.float32),
            jax.ShapeDtypeStruct((B, ED), jnp.float32),
        ],
        compiler_params=pltpu.CompilerParams(
            dimension_semantics=("parallel",)),
    )(fc, fl, pair, par, wd_t, bd, wc_t, wl_t, we_t, bs, wcls_t, bcls)


def kernel(feat_comp, feat_loc, id_loc, emb, W_deep, b_deep,
           W_shared, b_shared, W_cls, b_cls):
    vocab, d = emb.shape
    ids = id_loc.astype(jnp.int32)
    pair_idx = lax.shift_right_logical(ids, 1)
    parity = (ids & 1).reshape(B, 1)
    table = emb.reshape(vocab // 2, 2 * d)
    pair_rows = _make_gather(vocab // 2, 2 * d, B)(table, pair_idx)

    wd_t = W_deep.T
    wc_t = W_shared[:, :FC].T
    wl_t = W_shared[:, FC:FC + FL].T
    we_t = W_shared[:, FC + FL:].T
    wcls_t = W_cls.T

    outputs, wide_feat, embed_feat = _mlp(
        feat_comp, feat_loc, pair_rows, parity,
        wd_t, b_deep.reshape(1, -1),
        wc_t, wl_t, we_t, b_shared.reshape(1, -1),
        wcls_t, b_cls.reshape(1, -1))
    return (outputs, wide_feat, embed_feat)


# speedup vs baseline: 1.0143x; 1.0143x over previous
import functools

import jax
import jax.numpy as jnp
from jax import lax
from jax.experimental import pallas as pl
from jax.experimental.pallas import tpu as pltpu
from jax.experimental.pallas import tpu_sc as plsc

B = 16384
ED = 64
FC = 102
FL = 23

_NEG_SLOPE = 0.01


def _make_gather(rows: int, d: int, b: int):
    info = plsc.get_sparse_core_info()
    nw = info.num_cores * info.num_subcores
    nl = info.num_lanes
    assert d % nl == 0 and b % (8 * nw) == 0
    b_per_w = b // nw
    mesh = plsc.VectorSubcoreMesh(core_axis_name="c", subcore_axis_name="s")

    @functools.partial(
        pl.kernel,
        mesh=mesh,
        out_type=jax.ShapeDtypeStruct((b, d), jnp.float32),
        scratch_types=[
            pltpu.VMEM((b_per_w,), jnp.int32),
            pltpu.VMEM((b_per_w, d), jnp.float32),
            pltpu.SemaphoreType.DMA,
        ],
    )
    def gather_kernel(table_hbm, ids_hbm, out_hbm, idx_v, rows_v, sem):
        wid = lax.axis_index("s") * info.num_cores + lax.axis_index("c")
        base = wid * b_per_w
        pltpu.sync_copy(ids_hbm.at[pl.ds(base, b_per_w)], idx_v)
        for i in range(b_per_w // nl):
            sl = pl.ds(i * nl, nl)
            idx_v[sl] = jnp.right_shift(idx_v[sl], 1)
        pltpu.async_copy(table_hbm.at[idx_v], rows_v, sem).wait()
        pltpu.sync_copy(rows_v, out_hbm.at[pl.ds(base, b_per_w)])

    return gather_kernel


def _leaky(x):
    return jnp.where(x >= 0, x, _NEG_SLOPE * x)


def _mlp_body(fc_ref, fl_ref, pair_ref, ids_ref, wd_ref, bd_ref, wc_ref,
              wl_ref, we_ref, bs_ref, wcls_ref, bcls_ref,
              out_ref, wide_ref, ef_ref):
    fc = fc_ref[...]
    fl = fl_ref[...]
    pair = pair_ref[...]
    par = (ids_ref[...] & 1).astype(jnp.float32)
    lane = lax.broadcasted_iota(jnp.int32, pair.shape, 1)
    m = jnp.where(lane < ED, 1.0 - par, par)
    ef = _leaky(jnp.dot(pair * m, wd_ref[...],
                        preferred_element_type=jnp.float32) + bd_ref[...])
    h = (jnp.dot(fc, wc_ref[...], preferred_element_type=jnp.float32)
         + jnp.dot(fl, wl_ref[...], preferred_element_type=jnp.float32)
         + jnp.dot(ef, we_ref[...], preferred_element_type=jnp.float32)
         + bs_ref[...])
    h = _leaky(h)
    out_ref[...] = jnp.dot(h, wcls_ref[...], preferred_element_type=jnp.float32) + bcls_ref[...]
    wide_ref[...] = jnp.concatenate([fc, fl], axis=1)
    ef_ref[...] = ef


def _mlp(fc, fl, pair, ids2d, wd_t, bd, wc_t, wl_t, we_t, bs, wcls_t, bcls,
         tile: int = 2048):
    grid = (B // tile,)
    row = lambda i: (i, 0)
    rep = lambda i: (0, 0)
    return pl.pallas_call(
        _mlp_body,
        grid=grid,
        in_specs=[
            pl.BlockSpec((tile, FC), row),
            pl.BlockSpec((tile, FL), row),
            pl.BlockSpec((tile, 2 * ED), row),
            pl.BlockSpec((tile, 1), row),
            pl.BlockSpec(wd_t.shape, rep),
            pl.BlockSpec(bd.shape, rep),
            pl.BlockSpec(wc_t.shape, rep),
            pl.BlockSpec(wl_t.shape, rep),
            pl.BlockSpec(we_t.shape, rep),
            pl.BlockSpec(bs.shape, rep),
            pl.BlockSpec(wcls_t.shape, rep),
            pl.BlockSpec(bcls.shape, rep),
        ],
        out_specs=[
            pl.BlockSpec((tile, 2), row),
            pl.BlockSpec((tile, FC + FL), row),
            pl.BlockSpec((tile, ED), row),
        ],
        out_shape=[
            jax.ShapeDtypeStruct((B, 2), jnp.float32),
            jax.ShapeDtypeStruct((B, FC + FL), jnp.float32),
            jax.ShapeDtypeStruct((B, ED), jnp.float32),
        ],
        compiler_params=pltpu.CompilerParams(
            dimension_semantics=("parallel",)),
    )(fc, fl, pair, ids2d, wd_t, bd, wc_t, wl_t, we_t, bs, wcls_t, bcls)


def kernel(feat_comp, feat_loc, id_loc, emb, W_deep, b_deep,
           W_shared, b_shared, W_cls, b_cls):
    vocab, d = emb.shape
    ids = id_loc.astype(jnp.int32)
    table = emb.reshape(vocab // 2, 2 * d)
    pair_rows = _make_gather(vocab // 2, 2 * d, B)(table, ids)

    wd_t = jnp.concatenate([W_deep.T, W_deep.T], axis=0)
    wc_t = W_shared[:, :FC].T
    wl_t = W_shared[:, FC:FC + FL].T
    we_t = W_shared[:, FC + FL:].T
    wcls_t = W_cls.T

    outputs, wide_feat, embed_feat = _mlp(
        feat_comp, feat_loc, pair_rows, ids.reshape(B, 1),
        wd_t, b_deep.reshape(1, -1),
        wc_t, wl_t, we_t, b_shared.reshape(1, -1),
        wcls_t, b_cls.reshape(1, -1))
    return (outputs, wide_feat, embed_feat)


# speedup vs baseline: 1.0312x; 1.0166x over previous
import functools

import jax
import jax.numpy as jnp
from jax import lax
from jax.experimental import pallas as pl
from jax.experimental.pallas import tpu as pltpu
from jax.experimental.pallas import tpu_sc as plsc

B = 16384
ED = 64
FC = 102
FL = 23

_NEG_SLOPE = 0.01


def _make_gather(vocab: int, d: int, b: int):
    info = plsc.get_sparse_core_info()
    nw = info.num_cores * info.num_subcores
    assert d % info.num_lanes == 0 and b % (8 * nw) == 0
    b_per_w = b // nw
    mesh = plsc.VectorSubcoreMesh(core_axis_name="c", subcore_axis_name="s")

    @functools.partial(
        pl.kernel,
        mesh=mesh,
        out_type=jax.ShapeDtypeStruct((b, d), jnp.float32),
        scratch_types=[
            pltpu.VMEM((b_per_w,), jnp.float32),
            pltpu.VMEM((b_per_w,), jnp.int32),
            pltpu.VMEM((b_per_w, d), jnp.float32),
            pltpu.SemaphoreType.DMA,
        ],
        compiler_params=pltpu.CompilerParams(use_tc_tiling_on_sc=False, needs_layout_passes=False),
    )
    def gather_kernel(table_hbm, idxf_hbm, out_hbm, idx_vf, idx_v, rows_v, sem):
        wid = lax.axis_index("s") * info.num_cores + lax.axis_index("c")
        base = wid * b_per_w
        pltpu.sync_copy(idxf_hbm.at[pl.ds(base, b_per_w)], idx_vf)
        for i in range(b_per_w // info.num_lanes):
            sl = pl.ds(i * info.num_lanes, info.num_lanes)
            idx_v[sl] = plsc.bitcast(idx_vf[sl], jnp.int32)
        pltpu.async_copy(table_hbm.at[idx_v], rows_v, sem).wait()
        pltpu.sync_copy(rows_v, out_hbm.at[pl.ds(base, b_per_w)])

    return gather_kernel


def _leaky(x):
    return jnp.where(x >= 0, x, _NEG_SLOPE * x)


def _mlp_body(fc_ref, fl_ref, er_ref, wd_ref, bd_ref, wc_ref, wl_ref,
              we_ref, bs_ref, wcls_ref, bcls_ref,
              out_ref, wide_ref, ef_ref):
    fc = fc_ref[...]
    fl = fl_ref[...]
    er = er_ref[...]
    ef = _leaky(jnp.dot(er, wd_ref[...], preferred_element_type=jnp.float32)
                + bd_ref[...])
    h = (jnp.dot(fc, wc_ref[...], preferred_element_type=jnp.float32)
         + jnp.dot(fl, wl_ref[...], preferred_element_type=jnp.float32)
         + jnp.dot(ef, we_ref[...], preferred_element_type=jnp.float32)
         + bs_ref[...])
    h = _leaky(h)
    out_ref[...] = jnp.dot(h, wcls_ref[...], preferred_element_type=jnp.float32) + bcls_ref[...]
    wide_ref[...] = jnp.concatenate([fc, fl], axis=1)
    ef_ref[...] = ef


def _mlp(fc, fl, emb_rows, wd_t, bd, wc_t, wl_t, we_t, bs, wcls_t, bcls,
         tile: int = 4096):
    grid = (B // tile,)
    row = lambda i: (i, 0)
    rep = lambda i: (0, 0)
    return pl.pallas_call(
        _mlp_body,
        grid=grid,
        in_specs=[
            pl.BlockSpec((tile, FC), row),
            pl.BlockSpec((tile, FL), row),
            pl.BlockSpec((tile, ED), row),
            pl.BlockSpec(wd_t.shape, rep),
            pl.BlockSpec(bd.shape, rep),
            pl.BlockSpec(wc_t.shape, rep),
            pl.BlockSpec(wl_t.shape, rep),
            pl.BlockSpec(we_t.shape, rep),
            pl.BlockSpec(bs.shape, rep),
            pl.BlockSpec(wcls_t.shape, rep),
            pl.BlockSpec(bcls.shape, rep),
        ],
        out_specs=[
            pl.BlockSpec((tile, 2), row),
            pl.BlockSpec((tile, FC + FL), row),
            pl.BlockSpec((tile, ED), row),
        ],
        out_shape=[
            jax.ShapeDtypeStruct((B, 2), jnp.float32),
            jax.ShapeDtypeStruct((B, FC + FL), jnp.float32),
            jax.ShapeDtypeStruct((B, ED), jnp.float32),
        ],
        compiler_params=pltpu.CompilerParams(
            dimension_semantics=("parallel",)),
    )(fc, fl, emb_rows, wd_t, bd, wc_t, wl_t, we_t, bs, wcls_t, bcls)


def kernel(feat_comp, feat_loc, id_loc, emb, W_deep, b_deep,
           W_shared, b_shared, W_cls, b_cls):
    vocab, d = emb.shape
    ids_f = lax.bitcast_convert_type(id_loc.astype(jnp.int32), jnp.float32)
    emb_rows = _make_gather(vocab, d, B)(emb, ids_f)

    wd_t = W_deep.T
    wc_t = W_shared[:, :FC].T
    wl_t = W_shared[:, FC:FC + FL].T
    we_t = W_shared[:, FC + FL:].T
    wcls_t = W_cls.T

    outputs, wide_feat, embed_feat = _mlp(
        feat_comp, feat_loc, emb_rows,
        wd_t, b_deep.reshape(1, -1),
        wc_t, wl_t, we_t, b_shared.reshape(1, -1),
        wcls_t, b_cls.reshape(1, -1))
    return (outputs, wide_feat, embed_feat)


# speedup vs baseline: 1.0393x; 1.0079x over previous
import functools

import jax
import jax.numpy as jnp
from jax import lax
from jax.experimental import pallas as pl
from jax.experimental.pallas import tpu as pltpu
from jax.experimental.pallas import tpu_sc as plsc

B = 16384
ED = 64
FC = 102
FL = 23

_NEG_SLOPE = 0.01


def _make_gather(vocab: int, d: int, b: int):
    info = plsc.get_sparse_core_info()
    nw = info.num_cores * info.num_subcores
    assert d % info.num_lanes == 0 and b % (8 * nw) == 0
    b_per_w = b // nw
    mesh = plsc.VectorSubcoreMesh(core_axis_name="c", subcore_axis_name="s")

    @functools.partial(
        pl.kernel,
        mesh=mesh,
        out_type=jax.ShapeDtypeStruct((b, d), jnp.float32),
        scratch_types=[
            pltpu.VMEM((b_per_w,), jnp.float32),
            pltpu.VMEM((b_per_w,), jnp.int32),
            pltpu.VMEM((b_per_w, d), jnp.float32),
            pltpu.SemaphoreType.DMA,
        ],
        compiler_params=pltpu.CompilerParams(use_tc_tiling_on_sc=False, needs_layout_passes=False),
    )
    def gather_kernel(table_hbm, idxf_hbm, out_hbm, idx_vf, idx_v, rows_v, sem):
        wid = lax.axis_index("s") * info.num_cores + lax.axis_index("c")
        base = wid * b_per_w
        pltpu.sync_copy(idxf_hbm.at[pl.ds(base, b_per_w)], idx_vf)
        for i in range(b_per_w // info.num_lanes):
            sl = pl.ds(i * info.num_lanes, info.num_lanes)
            idx_v[sl] = plsc.bitcast(idx_vf[sl], jnp.int32)
        pltpu.async_copy(table_hbm.at[idx_v], rows_v, sem).wait()
        pltpu.sync_copy(rows_v, out_hbm.at[pl.ds(base, b_per_w)])

    return gather_kernel


def _leaky(x):
    return jnp.where(x >= 0, x, _NEG_SLOPE * x)


def _mlp_body(fc_ref, fl_ref, er_ref, wd_ref, bd_ref, wc_ref, wl_ref,
              we_ref, bs_ref, wcls_ref, bcls_ref,
              out_ref, wide_ref, ef_ref):
    fc = fc_ref[...]
    fl = fl_ref[...]
    er = er_ref[...]
    ef = _leaky(jnp.dot(er, wd_ref[...], preferred_element_type=jnp.float32)
                + bd_ref[...])
    h = (jnp.dot(fc, wc_ref[...], preferred_element_type=jnp.float32)
         + jnp.dot(fl, wl_ref[...], preferred_element_type=jnp.float32)
         + jnp.dot(ef, we_ref[...], preferred_element_type=jnp.float32)
         + bs_ref[...])
    h = _leaky(h)
    out_ref[...] = jnp.dot(h, wcls_ref[...], preferred_element_type=jnp.float32) + bcls_ref[...]
    wide_ref[...] = jnp.concatenate([fc, fl], axis=1)
    ef_ref[...] = ef


def _mlp(fc, fl, emb_rows, wd_t, bd, wc_t, wl_t, we_t, bs, wcls_t, bcls,
         tile: int = 8192):
    grid = (B // tile,)
    row = lambda i: (i, 0)
    rep = lambda i: (0, 0)
    return pl.pallas_call(
        _mlp_body,
        grid=grid,
        in_specs=[
            pl.BlockSpec((tile, FC), row),
            pl.BlockSpec((tile, FL), row),
            pl.BlockSpec((tile, ED), row),
            pl.BlockSpec(wd_t.shape, rep),
            pl.BlockSpec(bd.shape, rep),
            pl.BlockSpec(wc_t.shape, rep),
            pl.BlockSpec(wl_t.shape, rep),
            pl.BlockSpec(we_t.shape, rep),
            pl.BlockSpec(bs.shape, rep),
            pl.BlockSpec(wcls_t.shape, rep),
            pl.BlockSpec(bcls.shape, rep),
        ],
        out_specs=[
            pl.BlockSpec((tile, 2), row),
            pl.BlockSpec((tile, FC + FL), row),
            pl.BlockSpec((tile, ED), row),
        ],
        out_shape=[
            jax.ShapeDtypeStruct((B, 2), jnp.float32),
            jax.ShapeDtypeStruct((B, FC + FL), jnp.float32),
            jax.ShapeDtypeStruct((B, ED), jnp.float32),
        ],
        compiler_params=pltpu.CompilerParams(
            dimension_semantics=("parallel",)),
    )(fc, fl, emb_rows, wd_t, bd, wc_t, wl_t, we_t, bs, wcls_t, bcls)


def kernel(feat_comp, feat_loc, id_loc, emb, W_deep, b_deep,
           W_shared, b_shared, W_cls, b_cls):
    vocab, d = emb.shape
    ids_f = lax.bitcast_convert_type(id_loc.astype(jnp.int32), jnp.float32)
    emb_rows = _make_gather(vocab, d, B)(emb, ids_f)

    wd_t = W_deep.T
    wc_t = W_shared[:, :FC].T
    wl_t = W_shared[:, FC:FC + FL].T
    we_t = W_shared[:, FC + FL:].T
    wcls_t = W_cls.T

    outputs, wide_feat, embed_feat = _mlp(
        feat_comp, feat_loc, emb_rows,
        wd_t, b_deep.reshape(1, -1),
        wc_t, wl_t, we_t, b_shared.reshape(1, -1),
        wcls_t, b_cls.reshape(1, -1))
    return (outputs, wide_feat, embed_feat)
